# baseline (speedup 1.0000x reference)
import jax
import jax.numpy as jnp
from jax.experimental import pallas as pl
from jax.experimental.pallas import tpu as pltpu

N = 10000
BM = 400


def _mm_body(x_ref, w_ref, o_ref):
    o_ref[...] = jnp.dot(x_ref[...], w_ref[...],
                         preferred_element_type=jnp.float32)


def _layer_body(a_ref, s_ref, w_ref, o_ref):
    acc = jnp.dot(a_ref[...], s_ref[...], preferred_element_type=jnp.float32)
    h = jnp.maximum(acc, 0.0)
    o_ref[...] = jnp.dot(h, w_ref[...], preferred_element_type=jnp.float32)


def _last_body(a_ref, s_ref, o_ref):
    acc = jnp.dot(a_ref[...], s_ref[...], preferred_element_type=jnp.float32)
    h = jnp.maximum(acc, 0.0)
    m = jnp.max(h, axis=-1, keepdims=True)
    e = jnp.exp(h - m)
    o_ref[...] = e / jnp.sum(e, axis=-1, keepdims=True)


def _support(x, w):
    n, d_in = x.shape
    d_out = w.shape[1]
    return pl.pallas_call(
        _mm_body,
        out_shape=jax.ShapeDtypeStruct((n, d_out), jnp.float32),
    )(x, w)


def _layer(adj, s, w_next):
    d = s.shape[1]
    d2 = w_next.shape[1]
    grid = (N // BM,)
    return pl.pallas_call(
        _layer_body,
        grid=grid,
        in_specs=[
            pl.BlockSpec((BM, N), lambda i: (i, 0)),
            pl.BlockSpec((N, d), lambda i: (0, 0)),
            pl.BlockSpec((d, d2), lambda i: (0, 0)),
        ],
        out_specs=pl.BlockSpec((BM, d2), lambda i: (i, 0)),
        out_shape=jax.ShapeDtypeStruct((N, d2), jnp.float32),
        compiler_params=pltpu.CompilerParams(
            dimension_semantics=("parallel",),
        ),
    )(adj, s, w_next)


def _last_layer(adj, s):
    d = s.shape[1]
    grid = (N // BM,)
    return pl.pallas_call(
        _last_body,
        grid=grid,
        in_specs=[
            pl.BlockSpec((BM, N), lambda i: (i, 0)),
            pl.BlockSpec((N, d), lambda i: (0, 0)),
        ],
        out_specs=pl.BlockSpec((BM, d), lambda i: (i, 0)),
        out_shape=jax.ShapeDtypeStruct((N, d), jnp.float32),
        compiler_params=pltpu.CompilerParams(
            dimension_semantics=("parallel",),
        ),
    )(adj, s)


def kernel(input, adj, W1, W2, W3):
    s1 = _support(input, W1)
    s2 = _layer(adj, s1, W2)
    s3 = _layer(adj, s2, W3)
    return _last_layer(adj, s3)
